# baseline (speedup 1.0000x reference)
import jax
import jax.numpy as jnp
from jax import lax
from jax.experimental import pallas as pl
from jax.experimental.pallas import tpu as pltpu

_B, _NC, _C, _H, _W = 4, 2, 8, 512, 512
_NI = 16
_RAD = 10.0
_W_INST, _W_VAR, _W_SEED = 1.0, 10.0, 1.0
_BIG = 1e9

_HB = 128
_NB = _H // _HB
_P = _HB * _W
_M = _P // 8


def _stats_body(emb_ref, inst_ref, kf_ref, rect_ref,
                sums_s, colc_s, rowc_s):
    i = pl.program_id(1)

    @pl.when(i == 0)
    def _init():
        sums_s[...] = jnp.zeros_like(sums_s)
        colc_s[...] = jnp.zeros_like(colc_s)

    emb = emb_ref[0]
    inst = inst_ref[0, 0].reshape(1, _P)
    ids = lax.broadcasted_iota(jnp.int32, (_NI, 1), 0)
    maskf = (inst == ids).astype(jnp.float32)

    sums_s[...] += lax.dot_general(
        maskf, emb, (((1,), (1,)), ((), ())),
        preferred_element_type=jnp.float32)

    mask3 = maskf.reshape(_NI, _HB, _W)
    colc_s[...] += jnp.sum(mask3, axis=1)
    rowc_s[i] = jnp.sum(mask3, axis=2)

    @pl.when(i == _NB - 1)
    def _fin():
        colc = colc_s[...]
        cnt = jnp.sum(colc, axis=1, keepdims=True)
        keyf = sums_s[...] / cnt
        nrm = jnp.sqrt(jnp.sum(keyf * keyf, axis=1, keepdims=True))
        kf_ref[0] = keyf / jnp.maximum(nrm, 1e-12)

        xi = lax.broadcasted_iota(
            jnp.int32, (_NI, _W), 1).astype(jnp.float32)
        minx_ = jnp.min(jnp.where(colc > 0, xi, _BIG), axis=1, keepdims=True)
        maxx_ = jnp.max(jnp.where(colc > 0, xi, -_BIG), axis=1, keepdims=True)
        miny_ = jnp.full((_NI, 1), _BIG, jnp.float32)
        maxy_ = jnp.full((_NI, 1), -_BIG, jnp.float32)
        for q in range(_NB):
            rows = rowc_s[q]
            yq = (lax.broadcasted_iota(jnp.int32, (_NI, _HB), 1)
                  + q * _HB).astype(jnp.float32)
            miny_ = jnp.minimum(miny_, jnp.min(
                jnp.where(rows > 0, yq, _BIG), axis=1, keepdims=True))
            maxy_ = jnp.maximum(maxy_, jnp.max(
                jnp.where(rows > 0, yq, -_BIG), axis=1, keepdims=True))
        x1 = jnp.where(minx_ > _RAD, minx_ - _RAD, 0.0)
        x2 = jnp.where(maxx_ < _W - _RAD, maxx_ + _RAD, float(_W))
        y1 = jnp.where(miny_ > _RAD, miny_ - _RAD, 0.0)
        y2 = jnp.where(maxy_ < _H - _RAD, maxy_ + _RAD, float(_H))
        rect_ref[0] = jnp.concatenate([x1, x2, y1, y2], axis=1)


def _loss_body(emb_ref, inst_ref, seed_ref, lbl_ref, kf_ref, rect_ref,
               out_ref, cmat_s, intra_s, focal_s, acc_s):
    b = pl.program_id(0)
    i = pl.program_id(1)

    @pl.when((b == 0) & (i == 0))
    def _init_all():
        focal_s[0, 0] = 0.0
        acc_s[0, 0] = 0.0

    @pl.when(i == 0)
    def _init_b():
        cmat_s[...] = jnp.zeros_like(cmat_s)
        intra_s[0, 0] = 0.0

    emb = emb_ref[0]
    inst = inst_ref[0, 0].reshape(1, _P)
    ids = lax.broadcasted_iota(jnp.int32, (_NI, 1), 0)
    maskb = (inst == ids).astype(jnp.bfloat16)

    kf = kf_ref[0]
    kf_hi = kf.astype(jnp.bfloat16)
    kf_lo = (kf - kf_hi.astype(jnp.float32)).astype(jnp.bfloat16)
    gat = (lax.dot_general(kf_hi, maskb, (((0,), (0,)), ((), ())),
                           preferred_element_type=jnp.float32)
           + lax.dot_general(kf_lo, maskb, (((0,), (0,)), ((), ())),
                             preferred_element_type=jnp.float32))
    prod3 = (gat * emb).reshape(_C, 8, _M)
    dotp = jnp.sum(prod3, axis=0)
    emb3 = emb.reshape(_C, 8, _M)
    nb2 = jnp.sum(emb3 * emb3, axis=0)
    cos = dotp * lax.rsqrt(jnp.maximum(nb2, 1e-16))
    intra_s[0, 0] += jnp.sum(jnp.exp(1.0 - cos) - 1.0)

    rect = rect_ref[0]
    x1 = rect[:, 0:1]
    x2 = rect[:, 1:2]
    y1 = rect[:, 2:3]
    y2 = rect[:, 3:4]
    xi = lax.broadcasted_iota(jnp.int32, (_NI, _W), 1).astype(jnp.float32)
    yi = (lax.broadcasted_iota(jnp.int32, (_NI, _HB), 1)
          + i * _HB).astype(jnp.float32)
    winx = ((xi >= x1) & (xi < x2)).astype(jnp.bfloat16)
    winy = ((yi >= y1) & (yi < y2)).astype(jnp.bfloat16)
    win = (winy[:, :, None] * winx[:, None, :]).reshape(_NI, _P)
    cmat_s[...] += lax.dot_general(
        win, maskb, (((1,), (1,)), ((), ())),
        preferred_element_type=jnp.float32)

    s0 = seed_ref[0, 0]
    s1 = seed_ref[0, 1]
    lbl = lbl_ref[0]
    e0 = jnp.exp(s0)
    e1 = jnp.exp(s1)
    lse = jnp.log(e0 + e1)
    lpt = jnp.where(lbl == 0, s0, s1) - lse
    pt = jnp.exp(lpt)
    focal_s[0, 0] += jnp.sum((1.0 - pt) * (1.0 - pt) * (-lpt))

    @pl.when(i == _NB - 1)
    def _fin():
        kfv = kf_ref[0]
        norms = jnp.maximum(
            jnp.sqrt(jnp.sum(kfv * kfv, axis=1, keepdims=True)), 1e-8)
        gram = lax.dot_general(
            kfv, kfv, (((1,), (1,)), ((), ())),
            preferred_element_type=jnp.float32)
        outer = lax.dot_general(
            norms, norms, (((1,), (1,)), ((), ())),
            preferred_element_type=jnp.float32)
        s_abs = jnp.abs(gram / outer)

        present = (cmat_s[...] > 0.0).astype(jnp.float32)
        ki = lax.broadcasted_iota(jnp.int32, (_NI, _NI), 0)
        ji = lax.broadcasted_iota(jnp.int32, (_NI, _NI), 1)
        upper = (ki > ji).astype(jnp.float32)
        suf = lax.dot_general(
            present, upper, (((1,), (0,)), ((), ())),
            preferred_element_type=jnp.float32)
        rowm = jnp.where((ji >= 1) & (suf <= 9.0), present, 0.0)
        nm = jnp.where((ki == 0) | (ji == 0), 0.5, rowm)
        nm = jnp.where((ki == 0) & (ji == 0), 0.0, nm)
        inter = jnp.sum((jnp.exp(s_abs) - 1.0) * nm) / jnp.sum(nm)

        acc_s[0, 0] += inter * _W_INST + \
            (intra_s[0, 0] / float(_H * _W)) * _W_VAR
        out_ref[0, 0] = focal_s[0, 0] / float(_B * _H * _W) * _W_SEED + \
            acc_s[0, 0] / float(_B)


def kernel(seed_maps, emb_maps, labels, instances):
    hw = _H * _W
    emb2 = emb_maps.reshape(_B, _C, hw)
    inst2 = instances.reshape(_B, 1, hw)
    seed3 = seed_maps.reshape(_B, _NC, 8, hw // 8)
    lbl3 = labels.reshape(_B, 8, hw // 8)

    kf, rect = pl.pallas_call(
        _stats_body,
        grid=(_B, _NB),
        in_specs=[
            pl.BlockSpec((1, _C, _P), lambda b, i: (b, 0, i)),
            pl.BlockSpec((1, 1, _P), lambda b, i: (b, 0, i)),
        ],
        out_specs=[
            pl.BlockSpec((1, _NI, _C), lambda b, i: (b, 0, 0)),
            pl.BlockSpec((1, _NI, 4), lambda b, i: (b, 0, 0)),
        ],
        out_shape=[
            jax.ShapeDtypeStruct((_B, _NI, _C), jnp.float32),
            jax.ShapeDtypeStruct((_B, _NI, 4), jnp.float32),
        ],
        scratch_shapes=[
            pltpu.VMEM((_NI, _C), jnp.float32),
            pltpu.VMEM((_NI, _W), jnp.float32),
            pltpu.VMEM((_NB, _NI, _HB), jnp.float32),
        ],
    )(emb2, inst2)

    out = pl.pallas_call(
        _loss_body,
        grid=(_B, _NB),
        in_specs=[
            pl.BlockSpec((1, _C, _P), lambda b, i: (b, 0, i)),
            pl.BlockSpec((1, 1, _P), lambda b, i: (b, 0, i)),
            pl.BlockSpec((1, _NC, 8, _M), lambda b, i: (b, 0, 0, i)),
            pl.BlockSpec((1, 8, _M), lambda b, i: (b, 0, i)),
            pl.BlockSpec((1, _NI, _C), lambda b, i: (b, 0, 0)),
            pl.BlockSpec((1, _NI, 4), lambda b, i: (b, 0, 0)),
        ],
        out_specs=pl.BlockSpec(memory_space=pltpu.SMEM),
        out_shape=jax.ShapeDtypeStruct((1, 1), jnp.float32),
        scratch_shapes=[
            pltpu.VMEM((_NI, _NI), jnp.float32),
            pltpu.SMEM((1, 1), jnp.float32),
            pltpu.SMEM((1, 1), jnp.float32),
            pltpu.SMEM((1, 1), jnp.float32),
        ],
    )(emb2, inst2, seed3, lbl3, kf, rect)

    return out[0, 0]


# speedup vs baseline: 1.5766x; 1.5766x over previous
import jax
import jax.numpy as jnp
from jax import lax
from jax.experimental import pallas as pl
from jax.experimental.pallas import tpu as pltpu

_B, _NC, _C, _H, _W = 4, 2, 8, 512, 512
_NI = 16
_RAD = 10.0
_W_INST, _W_VAR, _W_SEED = 1.0, 10.0, 1.0
_BIG = 1e9

_HB = 128
_NB = _H // _HB
_P = _HB * _W
_M = _P // 8


def _body(emb_ref, inst_ref, seed_ref, lbl_ref, out_ref,
          embc_s, instc_s, sums_s, colc_s, rowc_s, kf_s, rect_s,
          cmat_s, intra_s, focal_s, acc_s):
    b = pl.program_id(0)
    ph = pl.program_id(1)
    i = pl.program_id(2)

    @pl.when((b == 0) & (ph == 0) & (i == 0))
    def _init_all():
        focal_s[0, 0] = 0.0
        acc_s[0, 0] = 0.0

    @pl.when((ph == 0) & (i == 0))
    def _init_b():
        sums_s[...] = jnp.zeros_like(sums_s)
        colc_s[...] = jnp.zeros_like(colc_s)
        cmat_s[...] = jnp.zeros_like(cmat_s)
        intra_s[0, 0] = 0.0

    ids = lax.broadcasted_iota(jnp.int32, (_NI, 1), 0)

    @pl.when(ph == 0)
    def _stats():
        emb = emb_ref[0].reshape(_C, _P)
        inst = inst_ref[0, 0].reshape(1, _P)
        embc_s[i] = emb
        instc_s[i] = inst
        maskf = (inst == ids).astype(jnp.float32)

        sums_s[...] += lax.dot_general(
            maskf, emb, (((1,), (1,)), ((), ())),
            preferred_element_type=jnp.float32)

        mask3 = maskf.reshape(_NI, _HB, _W)
        colc_s[...] += jnp.sum(mask3, axis=1)
        rowc_s[i] = jnp.sum(mask3, axis=2)

        @pl.when(i == _NB - 1)
        def _fin_stats():
            colc = colc_s[...]
            cnt = jnp.sum(colc, axis=1, keepdims=True)
            keyf = sums_s[...] / cnt
            nrm = jnp.sqrt(jnp.sum(keyf * keyf, axis=1, keepdims=True))
            kf_s[...] = keyf / jnp.maximum(nrm, 1e-12)

            xi = lax.broadcasted_iota(
                jnp.int32, (_NI, _W), 1).astype(jnp.float32)
            minx_ = jnp.min(
                jnp.where(colc > 0, xi, _BIG), axis=1, keepdims=True)
            maxx_ = jnp.max(
                jnp.where(colc > 0, xi, -_BIG), axis=1, keepdims=True)
            miny_ = jnp.full((_NI, 1), _BIG, jnp.float32)
            maxy_ = jnp.full((_NI, 1), -_BIG, jnp.float32)
            for q in range(_NB):
                rows = rowc_s[q]
                yq = (lax.broadcasted_iota(jnp.int32, (_NI, _HB), 1)
                      + q * _HB).astype(jnp.float32)
                miny_ = jnp.minimum(miny_, jnp.min(
                    jnp.where(rows > 0, yq, _BIG), axis=1, keepdims=True))
                maxy_ = jnp.maximum(maxy_, jnp.max(
                    jnp.where(rows > 0, yq, -_BIG), axis=1, keepdims=True))
            x1 = jnp.where(minx_ > _RAD, minx_ - _RAD, 0.0)
            x2 = jnp.where(maxx_ < _W - _RAD, maxx_ + _RAD, float(_W))
            y1 = jnp.where(miny_ > _RAD, miny_ - _RAD, 0.0)
            y2 = jnp.where(maxy_ < _H - _RAD, maxy_ + _RAD, float(_H))
            rect_s[...] = jnp.concatenate([x1, x2, y1, y2], axis=1)

    @pl.when(ph == 1)
    def _loss():
        emb = embc_s[i]
        inst = instc_s[i]
        maskb = (inst == ids).astype(jnp.bfloat16)

        kf = kf_s[...]
        kf_hi = kf.astype(jnp.bfloat16)
        kf_lo = (kf - kf_hi.astype(jnp.float32)).astype(jnp.bfloat16)
        gat = (lax.dot_general(kf_hi, maskb, (((0,), (0,)), ((), ())),
                               preferred_element_type=jnp.float32)
               + lax.dot_general(kf_lo, maskb, (((0,), (0,)), ((), ())),
                                 preferred_element_type=jnp.float32))
        prod3 = (gat * emb).reshape(_C, 8, _M)
        dotp = jnp.sum(prod3, axis=0)
        emb3 = emb.reshape(_C, 8, _M)
        nb2 = jnp.sum(emb3 * emb3, axis=0)
        cos = dotp * lax.rsqrt(jnp.maximum(nb2, 1e-16))
        intra_s[0, 0] += jnp.sum(jnp.exp(1.0 - cos) - 1.0)

        rect = rect_s[...]
        x1 = rect[:, 0:1]
        x2 = rect[:, 1:2]
        y1 = rect[:, 2:3]
        y2 = rect[:, 3:4]
        xi = lax.broadcasted_iota(
            jnp.int32, (_NI, _W), 1).astype(jnp.float32)
        yi = (lax.broadcasted_iota(jnp.int32, (_NI, _HB), 1)
              + i * _HB).astype(jnp.float32)
        winx = ((xi >= x1) & (xi < x2)).astype(jnp.bfloat16)
        winy = ((yi >= y1) & (yi < y2)).astype(jnp.bfloat16)
        win = (winy[:, :, None] * winx[:, None, :]).reshape(_NI, _P)
        cmat_s[...] += lax.dot_general(
            win, maskb, (((1,), (1,)), ((), ())),
            preferred_element_type=jnp.float32)

        s0 = seed_ref[0, 0]
        s1 = seed_ref[0, 1]
        lbl = lbl_ref[0]
        e0 = jnp.exp(s0)
        e1 = jnp.exp(s1)
        lse = jnp.log(e0 + e1)
        lpt = jnp.where(lbl == 0, s0, s1) - lse
        pt = jnp.exp(lpt)
        focal_s[0, 0] += jnp.sum((1.0 - pt) * (1.0 - pt) * (-lpt))

        @pl.when(i == _NB - 1)
        def _fin_loss():
            kfv = kf_s[...]
            norms = jnp.maximum(
                jnp.sqrt(jnp.sum(kfv * kfv, axis=1, keepdims=True)), 1e-8)
            gram = lax.dot_general(
                kfv, kfv, (((1,), (1,)), ((), ())),
                preferred_element_type=jnp.float32)
            outer = lax.dot_general(
                norms, norms, (((1,), (1,)), ((), ())),
                preferred_element_type=jnp.float32)
            s_abs = jnp.abs(gram / outer)

            present = (cmat_s[...] > 0.0).astype(jnp.float32)
            ki = lax.broadcasted_iota(jnp.int32, (_NI, _NI), 0)
            ji = lax.broadcasted_iota(jnp.int32, (_NI, _NI), 1)
            upper = (ki > ji).astype(jnp.float32)
            suf = lax.dot_general(
                present, upper, (((1,), (0,)), ((), ())),
                preferred_element_type=jnp.float32)
            rowm = jnp.where((ji >= 1) & (suf <= 9.0), present, 0.0)
            nm = jnp.where((ki == 0) | (ji == 0), 0.5, rowm)
            nm = jnp.where((ki == 0) & (ji == 0), 0.0, nm)
            inter = jnp.sum((jnp.exp(s_abs) - 1.0) * nm) / jnp.sum(nm)

            acc_s[0, 0] += inter * _W_INST + \
                (intra_s[0, 0] / float(_H * _W)) * _W_VAR
            out_ref[0, 0] = \
                focal_s[0, 0] / float(_B * _H * _W) * _W_SEED + \
                acc_s[0, 0] / float(_B)


def kernel(seed_maps, emb_maps, labels, instances):
    out = pl.pallas_call(
        _body,
        grid=(_B, 2, _NB),
        in_specs=[
            pl.BlockSpec((1, _C, _HB, _W),
                         lambda b, ph, i: (b, 0, i * (1 - ph)
                                           + (_NB - 1) * ph, 0)),
            pl.BlockSpec((1, 1, _HB, _W),
                         lambda b, ph, i: (b, 0, i * (1 - ph)
                                           + (_NB - 1) * ph, 0)),
            pl.BlockSpec((1, _NC, _HB, _W),
                         lambda b, ph, i: (b, 0, i * ph, 0)),
            pl.BlockSpec((1, _HB, _W), lambda b, ph, i: (b, i * ph, 0)),
        ],
        out_specs=pl.BlockSpec(memory_space=pltpu.SMEM),
        out_shape=jax.ShapeDtypeStruct((1, 1), jnp.float32),
        scratch_shapes=[
            pltpu.VMEM((_NB, _C, _P), jnp.float32),
            pltpu.VMEM((_NB, 1, _P), jnp.int32),
            pltpu.VMEM((_NI, _C), jnp.float32),
            pltpu.VMEM((_NI, _W), jnp.float32),
            pltpu.VMEM((_NB, _NI, _HB), jnp.float32),
            pltpu.VMEM((_NI, _C), jnp.float32),
            pltpu.VMEM((_NI, 4), jnp.float32),
            pltpu.VMEM((_NI, _NI), jnp.float32),
            pltpu.SMEM((1, 1), jnp.float32),
            pltpu.SMEM((1, 1), jnp.float32),
            pltpu.SMEM((1, 1), jnp.float32),
        ],
    )(emb_maps, instances, seed_maps, labels)

    return out[0, 0]


# speedup vs baseline: 1.6025x; 1.0164x over previous
import jax
import jax.numpy as jnp
from jax import lax
from jax.experimental import pallas as pl
from jax.experimental.pallas import tpu as pltpu

_B, _NC, _C, _H, _W = 4, 2, 8, 512, 512
_NI = 16
_RAD = 10.0
_W_INST, _W_VAR, _W_SEED = 1.0, 10.0, 1.0
_BIG = 1e9

_HB = 128
_NB = _H // _HB
_P = _HB * _W
_M = _P // 8


def _body(emb_ref, inst_ref, seed_ref, lbl_ref, out_ref,
          embc_s, maskc_s, sums_s, colc_s, rowc_s, kf_s, rect_s,
          cmat_s, intra_s, focal_s, acc_s):
    b = pl.program_id(0)
    ph = pl.program_id(1)
    i = pl.program_id(2)

    @pl.when((b == 0) & (ph == 0) & (i == 0))
    def _init_all():
        focal_s[0, 0] = 0.0
        acc_s[0, 0] = 0.0

    @pl.when((ph == 0) & (i == 0))
    def _init_b():
        sums_s[...] = jnp.zeros_like(sums_s)
        colc_s[...] = jnp.zeros_like(colc_s)
        cmat_s[...] = jnp.zeros_like(cmat_s)
        intra_s[0, 0] = 0.0

    ids = lax.broadcasted_iota(jnp.int32, (_NI, 1), 0)

    @pl.when(ph == 0)
    def _stats():
        emb = emb_ref[0].reshape(_C, _P)
        inst = inst_ref[0, 0].reshape(1, _P)
        embc_s[i] = emb
        maskf = (inst == ids).astype(jnp.float32)
        maskc_s[i] = maskf.astype(jnp.bfloat16)

        sums_s[...] += lax.dot_general(
            maskf, emb, (((1,), (1,)), ((), ())),
            preferred_element_type=jnp.float32)

        mask3 = maskf.reshape(_NI, _HB, _W)
        colc_s[...] += jnp.sum(mask3, axis=1)
        rowc_s[i] = jnp.sum(mask3, axis=2)

        @pl.when(i == _NB - 1)
        def _fin_stats():
            colc = colc_s[...]
            cnt = jnp.sum(colc, axis=1, keepdims=True)
            keyf = sums_s[...] / cnt
            nrm = jnp.sqrt(jnp.sum(keyf * keyf, axis=1, keepdims=True))
            kf_s[...] = keyf / jnp.maximum(nrm, 1e-12)

            xi = lax.broadcasted_iota(
                jnp.int32, (_NI, _W), 1).astype(jnp.float32)
            minx_ = jnp.min(
                jnp.where(colc > 0, xi, _BIG), axis=1, keepdims=True)
            maxx_ = jnp.max(
                jnp.where(colc > 0, xi, -_BIG), axis=1, keepdims=True)
            miny_ = jnp.full((_NI, 1), _BIG, jnp.float32)
            maxy_ = jnp.full((_NI, 1), -_BIG, jnp.float32)
            for q in range(_NB):
                rows = rowc_s[q]
                yq = (lax.broadcasted_iota(jnp.int32, (_NI, _HB), 1)
                      + q * _HB).astype(jnp.float32)
                miny_ = jnp.minimum(miny_, jnp.min(
                    jnp.where(rows > 0, yq, _BIG), axis=1, keepdims=True))
                maxy_ = jnp.maximum(maxy_, jnp.max(
                    jnp.where(rows > 0, yq, -_BIG), axis=1, keepdims=True))
            x1 = jnp.where(minx_ > _RAD, minx_ - _RAD, 0.0)
            x2 = jnp.where(maxx_ < _W - _RAD, maxx_ + _RAD, float(_W))
            y1 = jnp.where(miny_ > _RAD, miny_ - _RAD, 0.0)
            y2 = jnp.where(maxy_ < _H - _RAD, maxy_ + _RAD, float(_H))
            rect_s[...] = jnp.concatenate([x1, x2, y1, y2], axis=1)

    @pl.when(ph == 1)
    def _loss():
        emb = embc_s[i]
        maskb = maskc_s[i]

        kf = kf_s[...]
        kf_hi = kf.astype(jnp.bfloat16)
        kf_lo = (kf - kf_hi.astype(jnp.float32)).astype(jnp.bfloat16)
        gat = (lax.dot_general(kf_hi, maskb, (((0,), (0,)), ((), ())),
                               preferred_element_type=jnp.float32)
               + lax.dot_general(kf_lo, maskb, (((0,), (0,)), ((), ())),
                                 preferred_element_type=jnp.float32))
        prod3 = (gat * emb).reshape(_C, 8, _M)
        dotp = jnp.sum(prod3, axis=0)
        emb3 = emb.reshape(_C, 8, _M)
        nb2 = jnp.sum(emb3 * emb3, axis=0)
        cos = dotp * lax.rsqrt(jnp.maximum(nb2, 1e-16))
        intra_s[0, 0] += jnp.sum(jnp.exp(1.0 - cos) - 1.0)

        rect = rect_s[...]
        x1 = rect[:, 0:1]
        x2 = rect[:, 1:2]
        y1 = rect[:, 2:3]
        y2 = rect[:, 3:4]
        xi = lax.broadcasted_iota(
            jnp.int32, (_NI, _W), 1).astype(jnp.float32)
        yi = (lax.broadcasted_iota(jnp.int32, (_NI, _HB), 1)
              + i * _HB).astype(jnp.float32)
        winx = ((xi >= x1) & (xi < x2)).astype(jnp.bfloat16)
        winy = ((yi >= y1) & (yi < y2)).astype(jnp.bfloat16)
        win = (winy[:, :, None] * winx[:, None, :]).reshape(_NI, _P)
        cmat_s[...] += lax.dot_general(
            win, maskb, (((1,), (1,)), ((), ())),
            preferred_element_type=jnp.float32)

        s0 = seed_ref[0, 0]
        s1 = seed_ref[0, 1]
        lbl = lbl_ref[0]
        e0 = jnp.exp(s0)
        e1 = jnp.exp(s1)
        lse = jnp.log(e0 + e1)
        lpt = jnp.where(lbl == 0, s0, s1) - lse
        pt = jnp.exp(lpt)
        focal_s[0, 0] += jnp.sum((1.0 - pt) * (1.0 - pt) * (-lpt))

        @pl.when(i == _NB - 1)
        def _fin_loss():
            kfv = kf_s[...]
            norms = jnp.maximum(
                jnp.sqrt(jnp.sum(kfv * kfv, axis=1, keepdims=True)), 1e-8)
            gram = lax.dot_general(
                kfv, kfv, (((1,), (1,)), ((), ())),
                preferred_element_type=jnp.float32)
            outer = lax.dot_general(
                norms, norms, (((1,), (1,)), ((), ())),
                preferred_element_type=jnp.float32)
            s_abs = jnp.abs(gram / outer)

            present = (cmat_s[...] > 0.0).astype(jnp.float32)
            ki = lax.broadcasted_iota(jnp.int32, (_NI, _NI), 0)
            ji = lax.broadcasted_iota(jnp.int32, (_NI, _NI), 1)
            upper = (ki > ji).astype(jnp.float32)
            suf = lax.dot_general(
                present, upper, (((1,), (0,)), ((), ())),
                preferred_element_type=jnp.float32)
            rowm = jnp.where((ji >= 1) & (suf <= 9.0), present, 0.0)
            nm = jnp.where((ki == 0) | (ji == 0), 0.5, rowm)
            nm = jnp.where((ki == 0) & (ji == 0), 0.0, nm)
            inter = jnp.sum((jnp.exp(s_abs) - 1.0) * nm) / jnp.sum(nm)

            acc_s[0, 0] += inter * _W_INST + \
                (intra_s[0, 0] / float(_H * _W)) * _W_VAR
            out_ref[0, 0] = \
                focal_s[0, 0] / float(_B * _H * _W) * _W_SEED + \
                acc_s[0, 0] / float(_B)


def kernel(seed_maps, emb_maps, labels, instances):
    out = pl.pallas_call(
        _body,
        grid=(_B, 2, _NB),
        in_specs=[
            pl.BlockSpec((1, _C, _HB, _W),
                         lambda b, ph, i: (b, 0, i * (1 - ph)
                                           + (_NB - 1) * ph, 0)),
            pl.BlockSpec((1, 1, _HB, _W),
                         lambda b, ph, i: (b, 0, i * (1 - ph)
                                           + (_NB - 1) * ph, 0)),
            pl.BlockSpec((1, _NC, _HB, _W),
                         lambda b, ph, i: (b, 0, i * ph, 0)),
            pl.BlockSpec((1, _HB, _W), lambda b, ph, i: (b, i * ph, 0)),
        ],
        out_specs=pl.BlockSpec(memory_space=pltpu.SMEM),
        out_shape=jax.ShapeDtypeStruct((1, 1), jnp.float32),
        scratch_shapes=[
            pltpu.VMEM((_NB, _C, _P), jnp.float32),
            pltpu.VMEM((_NB, _NI, _P), jnp.bfloat16),
            pltpu.VMEM((_NI, _C), jnp.float32),
            pltpu.VMEM((_NI, _W), jnp.float32),
            pltpu.VMEM((_NB, _NI, _HB), jnp.float32),
            pltpu.VMEM((_NI, _C), jnp.float32),
            pltpu.VMEM((_NI, 4), jnp.float32),
            pltpu.VMEM((_NI, _NI), jnp.float32),
            pltpu.SMEM((1, 1), jnp.float32),
            pltpu.SMEM((1, 1), jnp.float32),
            pltpu.SMEM((1, 1), jnp.float32),
        ],
    )(emb_maps, instances, seed_maps, labels)

    return out[0, 0]


# speedup vs baseline: 1.6079x; 1.0034x over previous
import jax
import jax.numpy as jnp
from jax import lax
from jax.experimental import pallas as pl
from jax.experimental.pallas import tpu as pltpu

_B, _NC, _C, _H, _W = 4, 2, 8, 512, 512
_NI = 16
_RAD = 10.0
_W_INST, _W_VAR, _W_SEED = 1.0, 10.0, 1.0
_BIG = 1e9

_HB = 128
_NB = _H // _HB
_P = _HB * _W
_M = _P // 8


def _body(emb_ref, inst_ref, seed_ref, lbl_ref, out_ref,
          embc_s, maskc_s, sums_s, colc_s, rowc_s, kf_s, rect_s,
          cmat_s, intra_s, focal_s, acc_s):
    b = pl.program_id(0)
    ph = pl.program_id(1)
    i = pl.program_id(2)

    @pl.when((b == 0) & (ph == 0) & (i == 0))
    def _init_all():
        focal_s[0, 0] = 0.0
        acc_s[0, 0] = 0.0

    @pl.when((ph == 0) & (i == 0))
    def _init_b():
        sums_s[...] = jnp.zeros_like(sums_s)
        colc_s[...] = jnp.zeros_like(colc_s)
        cmat_s[...] = jnp.zeros_like(cmat_s)
        intra_s[0, 0] = 0.0

    ids = lax.broadcasted_iota(jnp.int32, (_NI, 1), 0)

    @pl.when(ph == 0)
    def _stats():
        emb = emb_ref[0].reshape(_C, _P)
        inst = inst_ref[0, 0].reshape(1, _P)
        embc_s[i] = emb
        maskf = (inst == ids).astype(jnp.float32)
        maskb = maskf.astype(jnp.bfloat16)
        maskc_s[i] = maskb

        sums_s[...] += lax.dot_general(
            maskf, emb, (((1,), (1,)), ((), ())),
            preferred_element_type=jnp.float32)

        mask3 = maskb.reshape(_NI, _HB, _W)
        colc_s[...] += jnp.sum(mask3, axis=1).astype(jnp.float32)
        rowc_s[i] = jnp.sum(mask3, axis=2).astype(jnp.float32)

        @pl.when(i == _NB - 1)
        def _fin_stats():
            colc = colc_s[...]
            cnt = jnp.sum(colc, axis=1, keepdims=True)
            keyf = sums_s[...] / cnt
            nrm = jnp.sqrt(jnp.sum(keyf * keyf, axis=1, keepdims=True))
            kf_s[...] = keyf / jnp.maximum(nrm, 1e-12)

            xi = lax.broadcasted_iota(
                jnp.int32, (_NI, _W), 1).astype(jnp.float32)
            minx_ = jnp.min(
                jnp.where(colc > 0, xi, _BIG), axis=1, keepdims=True)
            maxx_ = jnp.max(
                jnp.where(colc > 0, xi, -_BIG), axis=1, keepdims=True)
            miny_ = jnp.full((_NI, 1), _BIG, jnp.float32)
            maxy_ = jnp.full((_NI, 1), -_BIG, jnp.float32)
            for q in range(_NB):
                rows = rowc_s[q]
                yq = (lax.broadcasted_iota(jnp.int32, (_NI, _HB), 1)
                      + q * _HB).astype(jnp.float32)
                miny_ = jnp.minimum(miny_, jnp.min(
                    jnp.where(rows > 0, yq, _BIG), axis=1, keepdims=True))
                maxy_ = jnp.maximum(maxy_, jnp.max(
                    jnp.where(rows > 0, yq, -_BIG), axis=1, keepdims=True))
            x1 = jnp.where(minx_ > _RAD, minx_ - _RAD, 0.0)
            x2 = jnp.where(maxx_ < _W - _RAD, maxx_ + _RAD, float(_W))
            y1 = jnp.where(miny_ > _RAD, miny_ - _RAD, 0.0)
            y2 = jnp.where(maxy_ < _H - _RAD, maxy_ + _RAD, float(_H))
            rect_s[...] = jnp.concatenate([x1, x2, y1, y2], axis=1)

    @pl.when(ph == 1)
    def _loss():
        emb = embc_s[i]
        maskb = maskc_s[i]

        kf = kf_s[...]
        kf_hi = kf.astype(jnp.bfloat16)
        kf_lo = (kf - kf_hi.astype(jnp.float32)).astype(jnp.bfloat16)
        gat = (lax.dot_general(kf_hi, maskb, (((0,), (0,)), ((), ())),
                               preferred_element_type=jnp.float32)
               + lax.dot_general(kf_lo, maskb, (((0,), (0,)), ((), ())),
                                 preferred_element_type=jnp.float32))
        prod3 = (gat * emb).reshape(_C, 8, _M)
        dotp = jnp.sum(prod3, axis=0)
        emb3 = emb.reshape(_C, 8, _M)
        nb2 = jnp.sum(emb3 * emb3, axis=0)
        cos = dotp * lax.rsqrt(jnp.maximum(nb2, 1e-16))
        intra_s[0, 0] += jnp.sum(jnp.exp(1.0 - cos) - 1.0)

        rect = rect_s[...]
        x1 = rect[:, 0:1]
        x2 = rect[:, 1:2]
        y1 = rect[:, 2:3]
        y2 = rect[:, 3:4]
        xi = lax.broadcasted_iota(
            jnp.int32, (_NI, _W), 1).astype(jnp.float32)
        yi = (lax.broadcasted_iota(jnp.int32, (_NI, _HB), 1)
              + i * _HB).astype(jnp.float32)
        winx = ((xi >= x1) & (xi < x2)).astype(jnp.bfloat16)
        winy = ((yi >= y1) & (yi < y2)).astype(jnp.bfloat16)
        win = (winy[:, :, None] * winx[:, None, :]).reshape(_NI, _P)
        cmat_s[...] += lax.dot_general(
            win, maskb, (((1,), (1,)), ((), ())),
            preferred_element_type=jnp.float32)

        s0 = seed_ref[0, 0]
        s1 = seed_ref[0, 1]
        lbl = lbl_ref[0]
        e0 = jnp.exp(s0)
        e1 = jnp.exp(s1)
        lse = jnp.log(e0 + e1)
        lpt = jnp.where(lbl == 0, s0, s1) - lse
        pt = jnp.exp(lpt)
        focal_s[0, 0] += jnp.sum((1.0 - pt) * (1.0 - pt) * (-lpt))

        @pl.when(i == _NB - 1)
        def _fin_loss():
            kfv = kf_s[...]
            norms = jnp.maximum(
                jnp.sqrt(jnp.sum(kfv * kfv, axis=1, keepdims=True)), 1e-8)
            gram = lax.dot_general(
                kfv, kfv, (((1,), (1,)), ((), ())),
                preferred_element_type=jnp.float32)
            outer = lax.dot_general(
                norms, norms, (((1,), (1,)), ((), ())),
                preferred_element_type=jnp.float32)
            s_abs = jnp.abs(gram / outer)

            present = (cmat_s[...] > 0.0).astype(jnp.float32)
            ki = lax.broadcasted_iota(jnp.int32, (_NI, _NI), 0)
            ji = lax.broadcasted_iota(jnp.int32, (_NI, _NI), 1)
            upper = (ki > ji).astype(jnp.float32)
            suf = lax.dot_general(
                present, upper, (((1,), (0,)), ((), ())),
                preferred_element_type=jnp.float32)
            rowm = jnp.where((ji >= 1) & (suf <= 9.0), present, 0.0)
            nm = jnp.where((ki == 0) | (ji == 0), 0.5, rowm)
            nm = jnp.where((ki == 0) & (ji == 0), 0.0, nm)
            inter = jnp.sum((jnp.exp(s_abs) - 1.0) * nm) / jnp.sum(nm)

            acc_s[0, 0] += inter * _W_INST + \
                (intra_s[0, 0] / float(_H * _W)) * _W_VAR
            out_ref[0, 0] = \
                focal_s[0, 0] / float(_B * _H * _W) * _W_SEED + \
                acc_s[0, 0] / float(_B)


def kernel(seed_maps, emb_maps, labels, instances):
    out = pl.pallas_call(
        _body,
        grid=(_B, 2, _NB),
        in_specs=[
            pl.BlockSpec((1, _C, _HB, _W),
                         lambda b, ph, i: (b, 0, i * (1 - ph)
                                           + (_NB - 1) * ph, 0)),
            pl.BlockSpec((1, 1, _HB, _W),
                         lambda b, ph, i: (b, 0, i * (1 - ph)
                                           + (_NB - 1) * ph, 0)),
            pl.BlockSpec((1, _NC, _HB, _W),
                         lambda b, ph, i: (b, 0, i * ph, 0)),
            pl.BlockSpec((1, _HB, _W), lambda b, ph, i: (b, i * ph, 0)),
        ],
        out_specs=pl.BlockSpec(memory_space=pltpu.SMEM),
        out_shape=jax.ShapeDtypeStruct((1, 1), jnp.float32),
        scratch_shapes=[
            pltpu.VMEM((_NB, _C, _P), jnp.float32),
            pltpu.VMEM((_NB, _NI, _P), jnp.bfloat16),
            pltpu.VMEM((_NI, _C), jnp.float32),
            pltpu.VMEM((_NI, _W), jnp.float32),
            pltpu.VMEM((_NB, _NI, _HB), jnp.float32),
            pltpu.VMEM((_NI, _C), jnp.float32),
            pltpu.VMEM((_NI, 4), jnp.float32),
            pltpu.VMEM((_NI, _NI), jnp.float32),
            pltpu.SMEM((1, 1), jnp.float32),
            pltpu.SMEM((1, 1), jnp.float32),
            pltpu.SMEM((1, 1), jnp.float32),
        ],
    )(emb_maps, instances, seed_maps, labels)

    return out[0, 0]


# speedup vs baseline: 1.6434x; 1.0220x over previous
import jax
import jax.numpy as jnp
from jax import lax
from jax.experimental import pallas as pl
from jax.experimental.pallas import tpu as pltpu

_B, _NC, _C, _H, _W = 4, 2, 8, 512, 512
_NI = 16
_RAD = 10.0
_W_INST, _W_VAR, _W_SEED = 1.0, 10.0, 1.0
_BIG = 1e9

_HB = 256
_NB = _H // _HB
_P = _HB * _W
_M = _P // 8


def _body(emb_ref, inst_ref, seed_ref, lbl_ref, out_ref,
          embc_s, maskc_s, sums_s, colc_s, rowc_s, kf_s, rect_s,
          cmat_s, intra_s, focal_s, acc_s):
    b = pl.program_id(0)
    ph = pl.program_id(1)
    i = pl.program_id(2)

    @pl.when((b == 0) & (ph == 0) & (i == 0))
    def _init_all():
        focal_s[0, 0] = 0.0
        acc_s[0, 0] = 0.0

    @pl.when((ph == 0) & (i == 0))
    def _init_b():
        sums_s[...] = jnp.zeros_like(sums_s)
        colc_s[...] = jnp.zeros_like(colc_s)
        cmat_s[...] = jnp.zeros_like(cmat_s)
        intra_s[0, 0] = 0.0

    ids = lax.broadcasted_iota(jnp.int32, (_NI, 1), 0)

    @pl.when(ph == 0)
    def _stats():
        emb = emb_ref[0].reshape(_C, _P)
        inst = inst_ref[0, 0].reshape(1, _P)
        embc_s[i] = emb
        maskf = (inst == ids).astype(jnp.float32)
        maskb = maskf.astype(jnp.bfloat16)
        maskc_s[i] = maskb

        sums_s[...] += lax.dot_general(
            maskf, emb, (((1,), (1,)), ((), ())),
            preferred_element_type=jnp.float32)

        mask3 = maskb.reshape(_NI, _HB, _W)
        colc_s[...] += jnp.sum(mask3, axis=1).astype(jnp.float32)
        rowc_s[i] = jnp.sum(mask3, axis=2).astype(jnp.float32)

        @pl.when(i == _NB - 1)
        def _fin_stats():
            colc = colc_s[...]
            cnt = jnp.sum(colc, axis=1, keepdims=True)
            keyf = sums_s[...] / cnt
            nrm = jnp.sqrt(jnp.sum(keyf * keyf, axis=1, keepdims=True))
            kf_s[...] = keyf / jnp.maximum(nrm, 1e-12)

            xi = lax.broadcasted_iota(
                jnp.int32, (_NI, _W), 1).astype(jnp.float32)
            minx_ = jnp.min(
                jnp.where(colc > 0, xi, _BIG), axis=1, keepdims=True)
            maxx_ = jnp.max(
                jnp.where(colc > 0, xi, -_BIG), axis=1, keepdims=True)
            miny_ = jnp.full((_NI, 1), _BIG, jnp.float32)
            maxy_ = jnp.full((_NI, 1), -_BIG, jnp.float32)
            for q in range(_NB):
                rows = rowc_s[q]
                yq = (lax.broadcasted_iota(jnp.int32, (_NI, _HB), 1)
                      + q * _HB).astype(jnp.float32)
                miny_ = jnp.minimum(miny_, jnp.min(
                    jnp.where(rows > 0, yq, _BIG), axis=1, keepdims=True))
                maxy_ = jnp.maximum(maxy_, jnp.max(
                    jnp.where(rows > 0, yq, -_BIG), axis=1, keepdims=True))
            x1 = jnp.where(minx_ > _RAD, minx_ - _RAD, 0.0)
            x2 = jnp.where(maxx_ < _W - _RAD, maxx_ + _RAD, float(_W))
            y1 = jnp.where(miny_ > _RAD, miny_ - _RAD, 0.0)
            y2 = jnp.where(maxy_ < _H - _RAD, maxy_ + _RAD, float(_H))
            rect_s[...] = jnp.concatenate([x1, x2, y1, y2], axis=1)

    @pl.when(ph == 1)
    def _loss():
        emb = embc_s[i]
        maskb = maskc_s[i]

        kf = kf_s[...]
        kf_hi = kf.astype(jnp.bfloat16)
        kf_lo = (kf - kf_hi.astype(jnp.float32)).astype(jnp.bfloat16)
        gat = (lax.dot_general(kf_hi, maskb, (((0,), (0,)), ((), ())),
                               preferred_element_type=jnp.float32)
               + lax.dot_general(kf_lo, maskb, (((0,), (0,)), ((), ())),
                                 preferred_element_type=jnp.float32))
        prod3 = (gat * emb).reshape(_C, 8, _M)
        dotp = jnp.sum(prod3, axis=0)
        emb3 = emb.reshape(_C, 8, _M)
        nb2 = jnp.sum(emb3 * emb3, axis=0)
        cos = dotp * lax.rsqrt(jnp.maximum(nb2, 1e-16))
        intra_s[0, 0] += jnp.sum(jnp.exp(1.0 - cos) - 1.0)

        rect = rect_s[...]
        x1 = rect[:, 0:1]
        x2 = rect[:, 1:2]
        y1 = rect[:, 2:3]
        y2 = rect[:, 3:4]
        xi = lax.broadcasted_iota(
            jnp.int32, (_NI, _W), 1).astype(jnp.float32)
        yi = (lax.broadcasted_iota(jnp.int32, (_NI, _HB), 1)
              + i * _HB).astype(jnp.float32)
        winx = ((xi >= x1) & (xi < x2)).astype(jnp.bfloat16)
        winy = ((yi >= y1) & (yi < y2)).astype(jnp.bfloat16)
        win = (winy[:, :, None] * winx[:, None, :]).reshape(_NI, _P)
        cmat_s[...] += lax.dot_general(
            win, maskb, (((1,), (1,)), ((), ())),
            preferred_element_type=jnp.float32)

        s0 = seed_ref[0, 0]
        s1 = seed_ref[0, 1]
        lbl = lbl_ref[0]
        e0 = jnp.exp(s0)
        e1 = jnp.exp(s1)
        lse = jnp.log(e0 + e1)
        lpt = jnp.where(lbl == 0, s0, s1) - lse
        pt = jnp.exp(lpt)
        focal_s[0, 0] += jnp.sum((1.0 - pt) * (1.0 - pt) * (-lpt))

        @pl.when(i == _NB - 1)
        def _fin_loss():
            kfv = kf_s[...]
            norms = jnp.maximum(
                jnp.sqrt(jnp.sum(kfv * kfv, axis=1, keepdims=True)), 1e-8)
            gram = lax.dot_general(
                kfv, kfv, (((1,), (1,)), ((), ())),
                preferred_element_type=jnp.float32)
            outer = lax.dot_general(
                norms, norms, (((1,), (1,)), ((), ())),
                preferred_element_type=jnp.float32)
            s_abs = jnp.abs(gram / outer)

            present = (cmat_s[...] > 0.0).astype(jnp.float32)
            ki = lax.broadcasted_iota(jnp.int32, (_NI, _NI), 0)
            ji = lax.broadcasted_iota(jnp.int32, (_NI, _NI), 1)
            upper = (ki > ji).astype(jnp.float32)
            suf = lax.dot_general(
                present, upper, (((1,), (0,)), ((), ())),
                preferred_element_type=jnp.float32)
            rowm = jnp.where((ji >= 1) & (suf <= 9.0), present, 0.0)
            nm = jnp.where((ki == 0) | (ji == 0), 0.5, rowm)
            nm = jnp.where((ki == 0) & (ji == 0), 0.0, nm)
            inter = jnp.sum((jnp.exp(s_abs) - 1.0) * nm) / jnp.sum(nm)

            acc_s[0, 0] += inter * _W_INST + \
                (intra_s[0, 0] / float(_H * _W)) * _W_VAR
            out_ref[0, 0] = \
                focal_s[0, 0] / float(_B * _H * _W) * _W_SEED + \
                acc_s[0, 0] / float(_B)


def kernel(seed_maps, emb_maps, labels, instances):
    out = pl.pallas_call(
        _body,
        grid=(_B, 2, _NB),
        in_specs=[
            pl.BlockSpec((1, _C, _HB, _W),
                         lambda b, ph, i: (b, 0, i * (1 - ph)
                                           + (_NB - 1) * ph, 0)),
            pl.BlockSpec((1, 1, _HB, _W),
                         lambda b, ph, i: (b, 0, i * (1 - ph)
                                           + (_NB - 1) * ph, 0)),
            pl.BlockSpec((1, _NC, _HB, _W),
                         lambda b, ph, i: (b, 0, i * ph, 0)),
            pl.BlockSpec((1, _HB, _W), lambda b, ph, i: (b, i * ph, 0)),
        ],
        out_specs=pl.BlockSpec(memory_space=pltpu.SMEM),
        out_shape=jax.ShapeDtypeStruct((1, 1), jnp.float32),
        scratch_shapes=[
            pltpu.VMEM((_NB, _C, _P), jnp.float32),
            pltpu.VMEM((_NB, _NI, _P), jnp.bfloat16),
            pltpu.VMEM((_NI, _C), jnp.float32),
            pltpu.VMEM((_NI, _W), jnp.float32),
            pltpu.VMEM((_NB, _NI, _HB), jnp.float32),
            pltpu.VMEM((_NI, _C), jnp.float32),
            pltpu.VMEM((_NI, 4), jnp.float32),
            pltpu.VMEM((_NI, _NI), jnp.float32),
            pltpu.SMEM((1, 1), jnp.float32),
            pltpu.SMEM((1, 1), jnp.float32),
            pltpu.SMEM((1, 1), jnp.float32),
        ],
    )(emb_maps, instances, seed_maps, labels)

    return out[0, 0]
